# baseline (speedup 1.0000x reference)
import functools

import jax
import jax.numpy as jnp
from jax import lax
from jax.experimental import pallas as pl
from jax.experimental.pallas import tpu as pltpu
from jax.experimental.pallas import tpu_sc as plsc


def _make_sc_gather(V, D, B):
    info = plsc.get_sparse_core_info()
    NC, NS = info.num_cores, info.num_subcores
    NW = NC * NS
    assert B % NW == 0
    b_per_w = B // NW
    CH = 512
    assert b_per_w % CH == 0
    n_chunks = b_per_w // CH
    mesh = plsc.VectorSubcoreMesh(core_axis_name="c", subcore_axis_name="s")

    @functools.partial(
        pl.kernel,
        out_type=jax.ShapeDtypeStruct((B, D), jnp.float32),
        mesh=mesh,
        scratch_types=[
            pltpu.VMEM((CH,), jnp.int32),
            pltpu.VMEM((CH, D), jnp.float32),
            pltpu.SemaphoreType.DMA,
        ],
        compiler_params=pltpu.CompilerParams(use_tc_tiling_on_sc=False),
    )
    def sc_gather(table_hbm, idx_hbm, out_hbm, idx_v, rows_v, sem):
        wid = lax.axis_index("s") * NC + lax.axis_index("c")
        base = wid * b_per_w

        def body(c, carry):
            off = base + c * CH
            pltpu.sync_copy(idx_hbm.at[pl.ds(off, CH)], idx_v)
            pltpu.async_copy(table_hbm.at[idx_v], rows_v, sem).wait()
            pltpu.sync_copy(rows_v, out_hbm.at[pl.ds(off, CH)])
            return carry

        lax.fori_loop(0, n_chunks, body, 0)

    return sc_gather


def _make_tc_project(B, D_in, D_out):
    BM = 4096
    assert B % BM == 0

    def mm_body(x_ref, w_ref, b_ref, o_ref):
        o_ref[...] = (
            jnp.dot(x_ref[...], w_ref[...],
                    preferred_element_type=jnp.float32,
                    precision=lax.Precision.HIGHEST)
            + b_ref[...]
        )

    return pl.pallas_call(
        mm_body,
        grid=(B // BM,),
        in_specs=[
            pl.BlockSpec((BM, D_in), lambda i: (i, 0)),
            pl.BlockSpec((D_in, D_out), lambda i: (0, 0)),
            pl.BlockSpec((1, D_out), lambda i: (0, 0)),
        ],
        out_specs=pl.BlockSpec((BM, D_out), lambda i: (i, 0)),
        out_shape=jax.ShapeDtypeStruct((B, D_out), jnp.float32),
        compiler_params=pltpu.CompilerParams(
            dimension_semantics=("arbitrary",),
        ),
    )


def kernel(indices, table, W, b):
    Bt, L = indices.shape
    V, D = table.shape
    D_out = W.shape[1]
    Bf = Bt * L
    idx_flat = indices.reshape(Bf).astype(jnp.int32)
    gathered = _make_sc_gather(V, D, Bf)(table, idx_flat)
    out_flat = _make_tc_project(Bf, D, D_out)(gathered, W, b.reshape(1, D_out))
    return out_flat.reshape(Bt, L, D_out)
